# baseline (speedup 1.0000x reference)
import functools

import jax
import jax.numpy as jnp
import numpy as np
from jax import lax
from jax.experimental import pallas as pl
from jax.experimental.pallas import tpu as pltpu
from jax.experimental.pallas import tpu_sc as plsc

N = 10000
C = 128
H = C // 2
E = 320000
NC, NS = 2, 16
EPT = E // NS
G = 80
NBLK = EPT // G
NGRP = G // 16
NPAD = 10240
DPT = NPAD // NS
RPT = 624
BR = 1000
GRID = N // BR

_mesh = plsc.VectorSubcoreMesh(
    core_axis_name="c", subcore_axis_name="s", num_cores=NC, num_subcores=NS)

_GDN = lax.GatherDimensionNumbers(
    offset_dims=(), collapsed_slice_dims=(0,), start_index_map=(0,))


def _lane_bcast(v16, l):
    idx = jnp.full((16, 1), l, jnp.int32)
    return lax.gather(v16, idx, _GDN, (1,),
                      mode=lax.GatherScatterMode.PROMISE_IN_BOUNDS)


@functools.partial(
    pl.kernel,
    out_type=(
        jax.ShapeDtypeStruct((N, H), jnp.float32),
        jax.ShapeDtypeStruct((N, H), jnp.float32),
        jax.ShapeDtypeStruct((NPAD,), jnp.float32),
    ),
    mesh=_mesh,
    compiler_params=pltpu.CompilerParams(
        needs_layout_passes=False, use_tc_tiling_on_sc=False),
    scratch_types=[
        pltpu.VMEM((NBLK, G), jnp.int32),
        pltpu.VMEM((NBLK, G), jnp.int32),
        pltpu.VMEM((N,), jnp.float32),
        pltpu.VMEM((N,), jnp.float32),
        [pltpu.VMEM((G, H), jnp.float32) for _ in range(4)],
        [pltpu.VMEM((G,), jnp.float32) for _ in range(4)],
        pltpu.VMEM((DPT,), jnp.float32),
        [pltpu.SemaphoreType.DMA for _ in range(4)],
        [pltpu.SemaphoreType.DMA for _ in range(4)],
        [pltpu.SemaphoreType.DMA for _ in range(4)],
        pltpu.VMEM_SHARED((N, H), jnp.float32),
        pltpu.VMEM_SHARED((NPAD,), jnp.float32),
    ],
)
def _sc_edge(soff_hbm, dst_hbm, as_hbm, ad_hbm, h_hbm,
             out_lo, out_hi, den,
             soff_v, dst_v, as_v, ad_v, rows_v, ex_v, z_v,
             gsem, rsem, dsem, out_sm, den_sm):
    cid = lax.axis_index("c")
    sid = lax.axis_index("s")

    pltpu.sync_copy(soff_hbm.at[cid * NS + sid], soff_v)
    pltpu.sync_copy(dst_hbm.at[sid], dst_v)
    pltpu.sync_copy(as_hbm, as_v)
    pltpu.sync_copy(ad_hbm, ad_v)

    off16 = jnp.broadcast_to(cid * N, (16,)).astype(jnp.int32)

    zero16 = jnp.zeros((16,), jnp.float32)

    def zrow(i, carry):
        rows_v[0][i // (H // 16), pl.ds((i % (H // 16)) * 16, 16)] = zero16
        return carry
    lax.fori_loop(0, G * (H // 16), zrow, 0)

    def zden(i, carry):
        z_v[pl.ds(i * 16, 16)] = zero16
        return carry
    lax.fori_loop(0, DPT // 16, zden, 0)

    def zout(i, carry):
        pltpu.sync_copy(rows_v[0], out_sm.at[pl.ds(sid * RPT + i * G, G)])
        return carry
    lax.fori_loop(0, RPT // G, zout, 0)
    pltpu.sync_copy(rows_v[0].at[pl.ds(0, RPT % G)],
                    out_sm.at[pl.ds(sid * RPT + RPT - RPT % G, RPT % G)])

    @pl.when(sid == NS - 1)
    def _():
        pltpu.sync_copy(rows_v[0].at[pl.ds(0, N - NS * RPT)],
                        out_sm.at[pl.ds(NS * RPT, N - NS * RPT)])

    pltpu.sync_copy(z_v, den_sm.at[pl.ds(sid * DPT, DPT)])

    ninf = jnp.full((16,), -jnp.inf, jnp.float32)

    def mx(table):
        def body(i, m):
            return jnp.maximum(m, table[pl.ds(i * 16, 16)])
        return jnp.max(lax.fori_loop(0, N // 16, body, ninf))

    m_s = mx(as_v) + mx(ad_v)
    m_s = jnp.where(m_s >= 0.0, m_s, 0.2 * m_s)
    m16 = jnp.broadcast_to(m_s, (16,))

    plsc.subcore_barrier()

    def gather_h(blk, slot):
        pltpu.async_copy(h_hbm.at[soff_v.at[blk]], rows_v[slot], gsem[slot])

    def wait_gather(blk, slot):
        pltpu.make_async_copy(h_hbm.at[soff_v.at[0]], rows_v[slot],
                              gsem[slot]).wait()

    def start_scatter(blk, slot):
        pltpu.async_copy(rows_v[slot], out_sm.at[dst_v.at[blk]], rsem[slot],
                         add=True)

    def wait_scatter(blk, slot):
        pltpu.make_async_copy(rows_v[slot], out_sm.at[dst_v.at[0]],
                              rsem[slot]).wait()

    def start_dscatter(blk, slot):
        pltpu.async_copy(ex_v[slot], den_sm.at[dst_v.at[blk]], dsem[slot],
                         add=True)

    def wait_dscatter(blk, slot):
        pltpu.make_async_copy(ex_v[slot], den_sm.at[dst_v.at[0]],
                              dsem[slot]).wait()

    def compute(blk, slot):
        rv = rows_v[slot]
        ev = ex_v[slot]

        def grp_body(g, c2):
            s16 = soff_v[blk, pl.ds(g * 16, 16)] - off16
            d16 = dst_v[blk, pl.ds(g * 16, 16)]
            a = plsc.load_gather(as_v, [s16]) + plsc.load_gather(ad_v, [d16])
            a = jnp.where(a >= 0.0, a, 0.2 * a) - m16
            e16 = jnp.exp(a)
            ev[pl.ds(g * 16, 16)] = e16
            for l in range(16):
                w = _lane_bcast(e16, l)
                r = g * 16 + l
                for k in range(H // 16):
                    rv[r, pl.ds(k * 16, 16)] = rv[r, pl.ds(k * 16, 16)] * w
            return c2
        lax.fori_loop(0, NGRP, grp_body, 0)

    gather_h(0, 0)
    gather_h(1, 1)

    def quad(t, carry):
        for j in range(4):
            b = t * 4 + j

            @pl.when(b < NBLK)
            def _():
                slot = j
                nxt = (j + 2) % 4

                @pl.when(b >= 2)
                def _():
                    wait_scatter(b - 2, nxt)

                @pl.when(b + 2 < NBLK)
                def _():
                    gather_h(b + 2, nxt)

                @pl.when(jnp.logical_and(cid == 0, b >= 4))
                def _():
                    wait_dscatter(b - 4, slot)

                wait_gather(b, slot)
                compute(b, slot)
                start_scatter(b, slot)

                @pl.when(cid == 0)
                def _():
                    start_dscatter(b, slot)
        return carry
    lax.fori_loop(0, (NBLK + 3) // 4, quad, 0)

    wait_scatter(NBLK - 2, (NBLK - 2) % 4)
    wait_scatter(NBLK - 1, (NBLK - 1) % 4)

    @pl.when(cid == 0)
    def _():
        wait_dscatter(NBLK - 4, (NBLK - 4) % 4)
        wait_dscatter(NBLK - 3, (NBLK - 3) % 4)
        wait_dscatter(NBLK - 2, (NBLK - 2) % 4)
        wait_dscatter(NBLK - 1, (NBLK - 1) % 4)

    plsc.subcore_barrier()

    @pl.when(cid == 0)
    def _():
        pltpu.sync_copy(out_sm.at[pl.ds(sid * RPT, RPT)],
                        out_lo.at[pl.ds(sid * RPT, RPT)])
        pltpu.sync_copy(den_sm.at[pl.ds(sid * DPT, DPT)],
                        den.at[pl.ds(sid * DPT, DPT)])

        @pl.when(sid == NS - 1)
        def _():
            pltpu.sync_copy(out_sm.at[pl.ds(NS * RPT, N - NS * RPT)],
                            out_lo.at[pl.ds(NS * RPT, N - NS * RPT)])

    @pl.when(cid == 1)
    def _():
        pltpu.sync_copy(out_sm.at[pl.ds(sid * RPT, RPT)],
                        out_hi.at[pl.ds(sid * RPT, RPT)])

        @pl.when(sid == NS - 1)
        def _():
            pltpu.sync_copy(out_sm.at[pl.ds(NS * RPT, N - NS * RPT)],
                            out_hi.at[pl.ds(NS * RPT, N - NS * RPT)])


def _split_out(h, hlo_ref, hhi_ref, asrc, adst, as_ref, ad_ref):
    hlo_ref[...] = h[:, :H]
    hhi_ref[...] = h[:, H:]
    as_ref[...] = jnp.sum(h * asrc, axis=1, keepdims=True)
    ad_ref[...] = jnp.sum(h * adst, axis=1, keepdims=True)


_TRANSFORM_OUT_SPECS = [
    pl.BlockSpec((BR, H), lambda i: (i, 0)),
    pl.BlockSpec((BR, H), lambda i: (i, 0)),
    pl.BlockSpec((BR, 1), lambda i: (i, 0)),
    pl.BlockSpec((BR, 1), lambda i: (i, 0)),
]
_TRANSFORM_OUT_SHAPE = [
    jax.ShapeDtypeStruct((N, H), jnp.float32),
    jax.ShapeDtypeStruct((N, H), jnp.float32),
    jax.ShapeDtypeStruct((N, 1), jnp.float32),
    jax.ShapeDtypeStruct((N, 1), jnp.float32),
]


def _tc_transform_body(x_ref, w_ref, asrc_ref, adst_ref,
                       hlo_ref, hhi_ref, as_ref, ad_ref):
    h = jnp.dot(x_ref[...], w_ref[...], preferred_element_type=jnp.float32)
    _split_out(h, hlo_ref, hhi_ref, asrc_ref[...], adst_ref[...],
               as_ref, ad_ref)


def _tc_transform(x, w, asrc, adst):
    return pl.pallas_call(
        _tc_transform_body,
        grid=(GRID,),
        in_specs=[
            pl.BlockSpec((BR, C), lambda i: (i, 0)),
            pl.BlockSpec((C, C), lambda i: (0, 0)),
            pl.BlockSpec((1, C), lambda i: (0, 0)),
            pl.BlockSpec((1, C), lambda i: (0, 0)),
        ],
        out_specs=_TRANSFORM_OUT_SPECS,
        out_shape=_TRANSFORM_OUT_SHAPE,
    )(x, w, asrc, adst)


def _tc_combine_body(plo_ref, phi_ref, d_ref, b_ref,
                     w_ref, asrc_ref, adst_ref,
                     hlo_ref, hhi_ref, as_ref, ad_ref):
    den = d_ref[...] + 1e-16
    o = jnp.concatenate([plo_ref[...], phi_ref[...]], axis=1) / den + b_ref[...]
    o = jnp.where(o > 0.0, o, jnp.exp(jnp.minimum(o, 0.0)) - 1.0)
    h = jnp.dot(o, w_ref[...], preferred_element_type=jnp.float32)
    _split_out(h, hlo_ref, hhi_ref, asrc_ref[...], adst_ref[...],
               as_ref, ad_ref)


def _tc_combine(plo, phi, d, b, w, asrc, adst):
    return pl.pallas_call(
        _tc_combine_body,
        grid=(GRID,),
        in_specs=[
            pl.BlockSpec((BR, H), lambda i: (i, 0)),
            pl.BlockSpec((BR, H), lambda i: (i, 0)),
            pl.BlockSpec((BR, 1), lambda i: (i, 0)),
            pl.BlockSpec((1, C), lambda i: (0, 0)),
            pl.BlockSpec((C, C), lambda i: (0, 0)),
            pl.BlockSpec((1, C), lambda i: (0, 0)),
            pl.BlockSpec((1, C), lambda i: (0, 0)),
        ],
        out_specs=_TRANSFORM_OUT_SPECS,
        out_shape=_TRANSFORM_OUT_SHAPE,
    )(plo, phi, d, b, w, asrc, adst)


def _tc_final_body(plo_ref, phi_ref, d_ref, b_ref, o_ref):
    den = d_ref[...] + 1e-16
    o_ref[...] = (jnp.concatenate([plo_ref[...], phi_ref[...]], axis=1) / den
                  + b_ref[...])


def _tc_final(plo, phi, d, b):
    return pl.pallas_call(
        _tc_final_body,
        grid=(GRID,),
        in_specs=[
            pl.BlockSpec((BR, H), lambda i: (i, 0)),
            pl.BlockSpec((BR, H), lambda i: (i, 0)),
            pl.BlockSpec((BR, 1), lambda i: (i, 0)),
            pl.BlockSpec((1, C), lambda i: (0, 0)),
        ],
        out_specs=pl.BlockSpec((BR, C), lambda i: (i, 0)),
        out_shape=jax.ShapeDtypeStruct((N, C), jnp.float32),
    )(plo, phi, d, b)


def kernel(x, edge_index, W1, att_src1, att_dst1, b1,
           W2, att_src2, att_dst2, b2):
    ei = edge_index.astype(jnp.int32)
    src_r = ei[0].reshape(NS, NBLK, G)
    src = jnp.concatenate([src_r, src_r + N], axis=0)
    dst = ei[1].reshape(NS, NBLK, G)

    hlo1, hhi1, as1, ad1 = _tc_transform(
        x, W1, att_src1.reshape(1, C), att_dst1.reshape(1, C))
    olo, ohi, d1 = _sc_edge(src, dst, as1.reshape(-1), ad1.reshape(-1),
                            jnp.concatenate([hlo1, hhi1], axis=0))

    hlo2, hhi2, as2, ad2 = _tc_combine(
        olo, ohi, d1[:N].reshape(N, 1), b1.reshape(1, C),
        W2, att_src2.reshape(1, C), att_dst2.reshape(1, C))
    qlo, qhi, d2 = _sc_edge(src, dst, as2.reshape(-1), ad2.reshape(-1),
                            jnp.concatenate([hlo2, hhi2], axis=0))

    return _tc_final(qlo, qhi, d2[:N].reshape(N, 1), b2.reshape(1, C))


# speedup vs baseline: 1.6413x; 1.6413x over previous
import functools

import jax
import jax.numpy as jnp
import numpy as np
from jax import lax
from jax.experimental import pallas as pl
from jax.experimental.pallas import tpu as pltpu
from jax.experimental.pallas import tpu_sc as plsc

N = 10000
C = 128
H = C // 2
E = 320000
NC, NS = 2, 16
EPT = E // NS
G = 80
NBLK = EPT // G
NGRP = G // 16
NPAD = 10240
DPT = NPAD // NS
RPT = 624
BR = 1000
GRID = N // BR

_mesh = plsc.VectorSubcoreMesh(
    core_axis_name="c", subcore_axis_name="s", num_cores=NC, num_subcores=NS)

_GDN = lax.GatherDimensionNumbers(
    offset_dims=(), collapsed_slice_dims=(0,), start_index_map=(0,))


def _lane_bcast(v16, l):
    idx = jnp.full((16, 1), l, jnp.int32)
    return lax.gather(v16, idx, _GDN, (1,),
                      mode=lax.GatherScatterMode.PROMISE_IN_BOUNDS)


@functools.partial(
    pl.kernel,
    out_type=(
        jax.ShapeDtypeStruct((N, H), jnp.float32),
        jax.ShapeDtypeStruct((N, H), jnp.float32),
        jax.ShapeDtypeStruct((NPAD,), jnp.float32),
    ),
    mesh=_mesh,
    compiler_params=pltpu.CompilerParams(
        needs_layout_passes=False, use_tc_tiling_on_sc=False),
    scratch_types=[
        pltpu.VMEM((NBLK, G), jnp.int32),
        pltpu.VMEM((NBLK, G), jnp.int32),
        pltpu.VMEM((N,), jnp.float32),
        pltpu.VMEM((N,), jnp.float32),
        [pltpu.VMEM((G, H), jnp.float32) for _ in range(4)],
        [pltpu.VMEM((G,), jnp.float32) for _ in range(4)],
        pltpu.VMEM((DPT,), jnp.float32),
        [pltpu.SemaphoreType.DMA for _ in range(4)],
        [pltpu.SemaphoreType.DMA for _ in range(4)],
        [pltpu.SemaphoreType.DMA for _ in range(4)],
        pltpu.VMEM_SHARED((N, H), jnp.float32),
        pltpu.VMEM_SHARED((NPAD,), jnp.float32),
    ],
)
def _sc_edge(soff_hbm, dst_hbm, as_hbm, ad_hbm, h_hbm,
             out_lo, out_hi, den,
             soff_v, dst_v, as_v, ad_v, rows_v, ex_v, z_v,
             gsem, rsem, dsem, out_sm, den_sm):
    cid = lax.axis_index("c")
    sid = lax.axis_index("s")

    pltpu.sync_copy(soff_hbm.at[cid * NS + sid], soff_v)
    pltpu.sync_copy(dst_hbm.at[sid], dst_v)
    pltpu.sync_copy(as_hbm, as_v)
    pltpu.sync_copy(ad_hbm, ad_v)

    off16 = jnp.broadcast_to(cid * N, (16,)).astype(jnp.int32)

    zero16 = jnp.zeros((16,), jnp.float32)

    def zrow(i, carry):
        rows_v[0][i // (H // 16), pl.ds((i % (H // 16)) * 16, 16)] = zero16
        return carry
    lax.fori_loop(0, G * (H // 16), zrow, 0)

    def zden(i, carry):
        z_v[pl.ds(i * 16, 16)] = zero16
        return carry
    lax.fori_loop(0, DPT // 16, zden, 0)

    def zout(i, carry):
        pltpu.sync_copy(rows_v[0], out_sm.at[pl.ds(sid * RPT + i * G, G)])
        return carry
    lax.fori_loop(0, RPT // G, zout, 0)
    pltpu.sync_copy(rows_v[0].at[pl.ds(0, RPT % G)],
                    out_sm.at[pl.ds(sid * RPT + RPT - RPT % G, RPT % G)])

    @pl.when(sid == NS - 1)
    def _():
        pltpu.sync_copy(rows_v[0].at[pl.ds(0, N - NS * RPT)],
                        out_sm.at[pl.ds(NS * RPT, N - NS * RPT)])

    pltpu.sync_copy(z_v, den_sm.at[pl.ds(sid * DPT, DPT)])

    ninf = jnp.full((16,), -jnp.inf, jnp.float32)

    def mx(table):
        def body(i, m):
            return jnp.maximum(m, table[pl.ds(i * 16, 16)])
        return jnp.max(lax.fori_loop(0, N // 16, body, ninf))

    m_s = mx(as_v) + mx(ad_v)
    m_s = jnp.where(m_s >= 0.0, m_s, 0.2 * m_s)
    m16 = jnp.broadcast_to(m_s, (16,))

    plsc.subcore_barrier()

    def gather_h(blk, slot):
        pltpu.async_copy(h_hbm.at[soff_v.at[blk]], rows_v[slot], gsem[slot])

    def wait_gather(blk, slot):
        pltpu.make_async_copy(h_hbm.at[soff_v.at[0]], rows_v[slot],
                              gsem[slot]).wait()

    def start_scatter(blk, slot):
        pltpu.async_copy(rows_v[slot], out_sm.at[dst_v.at[blk]], rsem[slot],
                         add=True)

    def wait_scatter(blk, slot):
        pltpu.make_async_copy(rows_v[slot], out_sm.at[dst_v.at[0]],
                              rsem[slot]).wait()

    def start_dscatter(blk, slot):
        pltpu.async_copy(ex_v[slot], den_sm.at[dst_v.at[blk]], dsem[slot],
                         add=True)

    def wait_dscatter(blk, slot):
        pltpu.make_async_copy(ex_v[slot], den_sm.at[dst_v.at[0]],
                              dsem[slot]).wait()

    def compute(blk, slot):
        rv = rows_v[slot]
        ev = ex_v[slot]

        def att_body(g, c2):
            s16 = soff_v[blk, pl.ds(g * 16, 16)] - off16
            d16 = dst_v[blk, pl.ds(g * 16, 16)]
            a = plsc.load_gather(as_v, [s16]) + plsc.load_gather(ad_v, [d16])
            a = jnp.where(a >= 0.0, a, 0.2 * a) - m16
            ev[pl.ds(g * 16, 16)] = jnp.exp(a)
            return c2
        lax.fori_loop(0, NGRP, att_body, 0)

        def mul_body(e, c2):
            w = plsc.load_gather(ev, [jnp.full((16,), e, jnp.int32)])
            for k in range(H // 16):
                rv[e, pl.ds(k * 16, 16)] = rv[e, pl.ds(k * 16, 16)] * w
            return c2
        lax.fori_loop(0, G, mul_body, 0, unroll=4)

    gather_h(0, 0)
    gather_h(1, 1)

    def quad(t, carry):
        for j in range(4):
            b = t * 4 + j

            @pl.when(b < NBLK)
            def _():
                slot = j
                nxt = (j + 2) % 4

                @pl.when(b >= 2)
                def _():
                    wait_scatter(b - 2, nxt)

                @pl.when(b + 2 < NBLK)
                def _():
                    gather_h(b + 2, nxt)

                @pl.when(jnp.logical_and(cid == 0, b >= 4))
                def _():
                    wait_dscatter(b - 4, slot)

                wait_gather(b, slot)
                compute(b, slot)
                start_scatter(b, slot)

                @pl.when(cid == 0)
                def _():
                    start_dscatter(b, slot)
        return carry
    lax.fori_loop(0, (NBLK + 3) // 4, quad, 0)

    wait_scatter(NBLK - 2, (NBLK - 2) % 4)
    wait_scatter(NBLK - 1, (NBLK - 1) % 4)

    @pl.when(cid == 0)
    def _():
        wait_dscatter(NBLK - 4, (NBLK - 4) % 4)
        wait_dscatter(NBLK - 3, (NBLK - 3) % 4)
        wait_dscatter(NBLK - 2, (NBLK - 2) % 4)
        wait_dscatter(NBLK - 1, (NBLK - 1) % 4)

    plsc.subcore_barrier()

    @pl.when(cid == 0)
    def _():
        pltpu.sync_copy(out_sm.at[pl.ds(sid * RPT, RPT)],
                        out_lo.at[pl.ds(sid * RPT, RPT)])
        pltpu.sync_copy(den_sm.at[pl.ds(sid * DPT, DPT)],
                        den.at[pl.ds(sid * DPT, DPT)])

        @pl.when(sid == NS - 1)
        def _():
            pltpu.sync_copy(out_sm.at[pl.ds(NS * RPT, N - NS * RPT)],
                            out_lo.at[pl.ds(NS * RPT, N - NS * RPT)])

    @pl.when(cid == 1)
    def _():
        pltpu.sync_copy(out_sm.at[pl.ds(sid * RPT, RPT)],
                        out_hi.at[pl.ds(sid * RPT, RPT)])

        @pl.when(sid == NS - 1)
        def _():
            pltpu.sync_copy(out_sm.at[pl.ds(NS * RPT, N - NS * RPT)],
                            out_hi.at[pl.ds(NS * RPT, N - NS * RPT)])


def _split_out(h, hlo_ref, hhi_ref, asrc, adst, as_ref, ad_ref):
    hlo_ref[...] = h[:, :H]
    hhi_ref[...] = h[:, H:]
    as_ref[...] = jnp.sum(h * asrc, axis=1, keepdims=True)
    ad_ref[...] = jnp.sum(h * adst, axis=1, keepdims=True)


_TRANSFORM_OUT_SPECS = [
    pl.BlockSpec((BR, H), lambda i: (i, 0)),
    pl.BlockSpec((BR, H), lambda i: (i, 0)),
    pl.BlockSpec((BR, 1), lambda i: (i, 0)),
    pl.BlockSpec((BR, 1), lambda i: (i, 0)),
]
_TRANSFORM_OUT_SHAPE = [
    jax.ShapeDtypeStruct((N, H), jnp.float32),
    jax.ShapeDtypeStruct((N, H), jnp.float32),
    jax.ShapeDtypeStruct((N, 1), jnp.float32),
    jax.ShapeDtypeStruct((N, 1), jnp.float32),
]


def _tc_transform_body(x_ref, w_ref, asrc_ref, adst_ref,
                       hlo_ref, hhi_ref, as_ref, ad_ref):
    h = jnp.dot(x_ref[...], w_ref[...], preferred_element_type=jnp.float32)
    _split_out(h, hlo_ref, hhi_ref, asrc_ref[...], adst_ref[...],
               as_ref, ad_ref)


def _tc_transform(x, w, asrc, adst):
    return pl.pallas_call(
        _tc_transform_body,
        grid=(GRID,),
        in_specs=[
            pl.BlockSpec((BR, C), lambda i: (i, 0)),
            pl.BlockSpec((C, C), lambda i: (0, 0)),
            pl.BlockSpec((1, C), lambda i: (0, 0)),
            pl.BlockSpec((1, C), lambda i: (0, 0)),
        ],
        out_specs=_TRANSFORM_OUT_SPECS,
        out_shape=_TRANSFORM_OUT_SHAPE,
    )(x, w, asrc, adst)


def _tc_combine_body(plo_ref, phi_ref, d_ref, b_ref,
                     w_ref, asrc_ref, adst_ref,
                     hlo_ref, hhi_ref, as_ref, ad_ref):
    den = d_ref[...] + 1e-16
    o = jnp.concatenate([plo_ref[...], phi_ref[...]], axis=1) / den + b_ref[...]
    o = jnp.where(o > 0.0, o, jnp.exp(jnp.minimum(o, 0.0)) - 1.0)
    h = jnp.dot(o, w_ref[...], preferred_element_type=jnp.float32)
    _split_out(h, hlo_ref, hhi_ref, asrc_ref[...], adst_ref[...],
               as_ref, ad_ref)


def _tc_combine(plo, phi, d, b, w, asrc, adst):
    return pl.pallas_call(
        _tc_combine_body,
        grid=(GRID,),
        in_specs=[
            pl.BlockSpec((BR, H), lambda i: (i, 0)),
            pl.BlockSpec((BR, H), lambda i: (i, 0)),
            pl.BlockSpec((BR, 1), lambda i: (i, 0)),
            pl.BlockSpec((1, C), lambda i: (0, 0)),
            pl.BlockSpec((C, C), lambda i: (0, 0)),
            pl.BlockSpec((1, C), lambda i: (0, 0)),
            pl.BlockSpec((1, C), lambda i: (0, 0)),
        ],
        out_specs=_TRANSFORM_OUT_SPECS,
        out_shape=_TRANSFORM_OUT_SHAPE,
    )(plo, phi, d, b, w, asrc, adst)


def _tc_final_body(plo_ref, phi_ref, d_ref, b_ref, o_ref):
    den = d_ref[...] + 1e-16
    o_ref[...] = (jnp.concatenate([plo_ref[...], phi_ref[...]], axis=1) / den
                  + b_ref[...])


def _tc_final(plo, phi, d, b):
    return pl.pallas_call(
        _tc_final_body,
        grid=(GRID,),
        in_specs=[
            pl.BlockSpec((BR, H), lambda i: (i, 0)),
            pl.BlockSpec((BR, H), lambda i: (i, 0)),
            pl.BlockSpec((BR, 1), lambda i: (i, 0)),
            pl.BlockSpec((1, C), lambda i: (0, 0)),
        ],
        out_specs=pl.BlockSpec((BR, C), lambda i: (i, 0)),
        out_shape=jax.ShapeDtypeStruct((N, C), jnp.float32),
    )(plo, phi, d, b)


def kernel(x, edge_index, W1, att_src1, att_dst1, b1,
           W2, att_src2, att_dst2, b2):
    ei = edge_index.astype(jnp.int32)
    src_r = ei[0].reshape(NS, NBLK, G)
    src = jnp.concatenate([src_r, src_r + N], axis=0)
    dst = ei[1].reshape(NS, NBLK, G)

    hlo1, hhi1, as1, ad1 = _tc_transform(
        x, W1, att_src1.reshape(1, C), att_dst1.reshape(1, C))
    olo, ohi, d1 = _sc_edge(src, dst, as1.reshape(-1), ad1.reshape(-1),
                            jnp.concatenate([hlo1, hhi1], axis=0))

    hlo2, hhi2, as2, ad2 = _tc_combine(
        olo, ohi, d1[:N].reshape(N, 1), b1.reshape(1, C),
        W2, att_src2.reshape(1, C), att_dst2.reshape(1, C))
    qlo, qhi, d2 = _sc_edge(src, dst, as2.reshape(-1), ad2.reshape(-1),
                            jnp.concatenate([hlo2, hhi2], axis=0))

    return _tc_final(qlo, qhi, d2[:N].reshape(N, 1), b2.reshape(1, C))


# speedup vs baseline: 1.6435x; 1.0013x over previous
import functools

import jax
import jax.numpy as jnp
import numpy as np
from jax import lax
from jax.experimental import pallas as pl
from jax.experimental.pallas import tpu as pltpu
from jax.experimental.pallas import tpu_sc as plsc

N = 10000
C = 128
H = C // 2
E = 320000
NC, NS = 2, 16
EPT = E // NS
G = 80
NBLK = EPT // G
NGRP = G // 16
NPAD = 10240
DPT = NPAD // NS
RPT = 624
RING = 5
DEPTH = 3
BR = 1000
GRID = N // BR

_mesh = plsc.VectorSubcoreMesh(
    core_axis_name="c", subcore_axis_name="s", num_cores=NC, num_subcores=NS)

_GDN = lax.GatherDimensionNumbers(
    offset_dims=(), collapsed_slice_dims=(0,), start_index_map=(0,))


def _lane_bcast(v16, l):
    idx = jnp.full((16, 1), l, jnp.int32)
    return lax.gather(v16, idx, _GDN, (1,),
                      mode=lax.GatherScatterMode.PROMISE_IN_BOUNDS)


@functools.partial(
    pl.kernel,
    out_type=(
        jax.ShapeDtypeStruct((N, H), jnp.float32),
        jax.ShapeDtypeStruct((N, H), jnp.float32),
        jax.ShapeDtypeStruct((NPAD,), jnp.float32),
    ),
    mesh=_mesh,
    compiler_params=pltpu.CompilerParams(
        needs_layout_passes=False, use_tc_tiling_on_sc=False),
    scratch_types=[
        pltpu.VMEM((NBLK, G), jnp.int32),
        pltpu.VMEM((NBLK, G), jnp.int32),
        pltpu.VMEM((N,), jnp.float32),
        pltpu.VMEM((N,), jnp.float32),
        [pltpu.VMEM((G, H), jnp.float32) for _ in range(RING)],
        [pltpu.VMEM((G,), jnp.float32) for _ in range(RING)],
        pltpu.VMEM((DPT,), jnp.float32),
        [pltpu.SemaphoreType.DMA for _ in range(RING)],
        [pltpu.SemaphoreType.DMA for _ in range(RING)],
        [pltpu.SemaphoreType.DMA for _ in range(RING)],
        pltpu.VMEM_SHARED((N, H), jnp.float32),
        pltpu.VMEM_SHARED((NPAD,), jnp.float32),
    ],
)
def _sc_edge(soff_hbm, dst_hbm, as_hbm, ad_hbm, h_hbm,
             out_lo, out_hi, den,
             soff_v, dst_v, as_v, ad_v, rows_v, ex_v, z_v,
             gsem, rsem, dsem, out_sm, den_sm):
    cid = lax.axis_index("c")
    sid = lax.axis_index("s")

    pltpu.sync_copy(soff_hbm.at[cid * NS + sid], soff_v)
    pltpu.sync_copy(dst_hbm.at[sid], dst_v)
    pltpu.sync_copy(as_hbm, as_v)
    pltpu.sync_copy(ad_hbm, ad_v)

    off16 = jnp.broadcast_to(cid * N, (16,)).astype(jnp.int32)

    zero16 = jnp.zeros((16,), jnp.float32)

    def zrow(i, carry):
        rows_v[0][i // (H // 16), pl.ds((i % (H // 16)) * 16, 16)] = zero16
        return carry
    lax.fori_loop(0, G * (H // 16), zrow, 0)

    def zden(i, carry):
        z_v[pl.ds(i * 16, 16)] = zero16
        return carry
    lax.fori_loop(0, DPT // 16, zden, 0)

    def zout(i, carry):
        pltpu.sync_copy(rows_v[0], out_sm.at[pl.ds(sid * RPT + i * G, G)])
        return carry
    lax.fori_loop(0, RPT // G, zout, 0)
    pltpu.sync_copy(rows_v[0].at[pl.ds(0, RPT % G)],
                    out_sm.at[pl.ds(sid * RPT + RPT - RPT % G, RPT % G)])

    @pl.when(sid == NS - 1)
    def _():
        pltpu.sync_copy(rows_v[0].at[pl.ds(0, N - NS * RPT)],
                        out_sm.at[pl.ds(NS * RPT, N - NS * RPT)])

    pltpu.sync_copy(z_v, den_sm.at[pl.ds(sid * DPT, DPT)])

    ninf = jnp.full((16,), -jnp.inf, jnp.float32)

    def mx(table):
        def body(i, m):
            return jnp.maximum(m, table[pl.ds(i * 16, 16)])
        return jnp.max(lax.fori_loop(0, N // 16, body, ninf))

    m_s = mx(as_v) + mx(ad_v)
    m_s = jnp.where(m_s >= 0.0, m_s, 0.2 * m_s)
    m16 = jnp.broadcast_to(m_s, (16,))

    plsc.subcore_barrier()

    def gather_h(blk, slot):
        pltpu.async_copy(h_hbm.at[soff_v.at[blk]], rows_v[slot], gsem[slot])

    def wait_gather(blk, slot):
        pltpu.make_async_copy(h_hbm.at[soff_v.at[0]], rows_v[slot],
                              gsem[slot]).wait()

    def start_scatter(blk, slot):
        pltpu.async_copy(rows_v[slot], out_sm.at[dst_v.at[blk]], rsem[slot],
                         add=True)

    def wait_scatter(blk, slot):
        pltpu.make_async_copy(rows_v[slot], out_sm.at[dst_v.at[0]],
                              rsem[slot]).wait()

    def start_dscatter(blk, slot):
        pltpu.async_copy(ex_v[slot], den_sm.at[dst_v.at[blk]], dsem[slot],
                         add=True)

    def wait_dscatter(blk, slot):
        pltpu.make_async_copy(ex_v[slot], den_sm.at[dst_v.at[0]],
                              dsem[slot]).wait()

    def compute(blk, slot):
        rv = rows_v[slot]
        ev = ex_v[slot]

        def att_body(g, c2):
            s16 = soff_v[blk, pl.ds(g * 16, 16)] - off16
            d16 = dst_v[blk, pl.ds(g * 16, 16)]
            a = plsc.load_gather(as_v, [s16]) + plsc.load_gather(ad_v, [d16])
            a = jnp.where(a >= 0.0, a, 0.2 * a) - m16
            ev[pl.ds(g * 16, 16)] = jnp.exp(a)
            return c2
        lax.fori_loop(0, NGRP, att_body, 0)

        def mul_body(e, c2):
            w = plsc.load_gather(ev, [jnp.full((16,), e, jnp.int32)])
            for k in range(H // 16):
                rv[e, pl.ds(k * 16, 16)] = rv[e, pl.ds(k * 16, 16)] * w
            return c2
        lax.fori_loop(0, G, mul_body, 0, unroll=4)

    for p in range(DEPTH):
        gather_h(p, p)

    LAG = RING - DEPTH

    def rot(t, carry):
        for j in range(RING):
            b = t * RING + j

            @pl.when(b < NBLK)
            def _():
                slot = j
                nxt = (j + DEPTH) % RING

                @pl.when(b >= LAG)
                def _():
                    wait_scatter(b - LAG, nxt)

                @pl.when(b + DEPTH < NBLK)
                def _():
                    gather_h(b + DEPTH, nxt)

                @pl.when(jnp.logical_and(cid == 0, b >= RING))
                def _():
                    wait_dscatter(b - RING, slot)

                wait_gather(b, slot)
                compute(b, slot)
                start_scatter(b, slot)

                @pl.when(cid == 0)
                def _():
                    start_dscatter(b, slot)
        return carry
    lax.fori_loop(0, (NBLK + RING - 1) // RING, rot, 0)

    for i in range(LAG):
        wait_scatter(NBLK - LAG + i, (NBLK - LAG + i) % RING)

    @pl.when(cid == 0)
    def _():
        for i in range(RING):
            wait_dscatter(NBLK - RING + i, (NBLK - RING + i) % RING)

    plsc.subcore_barrier()

    @pl.when(cid == 0)
    def _():
        pltpu.sync_copy(out_sm.at[pl.ds(sid * RPT, RPT)],
                        out_lo.at[pl.ds(sid * RPT, RPT)])
        pltpu.sync_copy(den_sm.at[pl.ds(sid * DPT, DPT)],
                        den.at[pl.ds(sid * DPT, DPT)])

        @pl.when(sid == NS - 1)
        def _():
            pltpu.sync_copy(out_sm.at[pl.ds(NS * RPT, N - NS * RPT)],
                            out_lo.at[pl.ds(NS * RPT, N - NS * RPT)])

    @pl.when(cid == 1)
    def _():
        pltpu.sync_copy(out_sm.at[pl.ds(sid * RPT, RPT)],
                        out_hi.at[pl.ds(sid * RPT, RPT)])

        @pl.when(sid == NS - 1)
        def _():
            pltpu.sync_copy(out_sm.at[pl.ds(NS * RPT, N - NS * RPT)],
                            out_hi.at[pl.ds(NS * RPT, N - NS * RPT)])


def _split_out(h, hlo_ref, hhi_ref, asrc, adst, as_ref, ad_ref):
    hlo_ref[...] = h[:, :H]
    hhi_ref[...] = h[:, H:]
    as_ref[...] = jnp.sum(h * asrc, axis=1, keepdims=True)
    ad_ref[...] = jnp.sum(h * adst, axis=1, keepdims=True)


_TRANSFORM_OUT_SPECS = [
    pl.BlockSpec((BR, H), lambda i: (i, 0)),
    pl.BlockSpec((BR, H), lambda i: (i, 0)),
    pl.BlockSpec((BR, 1), lambda i: (i, 0)),
    pl.BlockSpec((BR, 1), lambda i: (i, 0)),
]
_TRANSFORM_OUT_SHAPE = [
    jax.ShapeDtypeStruct((N, H), jnp.float32),
    jax.ShapeDtypeStruct((N, H), jnp.float32),
    jax.ShapeDtypeStruct((N, 1), jnp.float32),
    jax.ShapeDtypeStruct((N, 1), jnp.float32),
]


def _tc_transform_body(x_ref, w_ref, asrc_ref, adst_ref,
                       hlo_ref, hhi_ref, as_ref, ad_ref):
    h = jnp.dot(x_ref[...], w_ref[...], preferred_element_type=jnp.float32)
    _split_out(h, hlo_ref, hhi_ref, asrc_ref[...], adst_ref[...],
               as_ref, ad_ref)


def _tc_transform(x, w, asrc, adst):
    return pl.pallas_call(
        _tc_transform_body,
        grid=(GRID,),
        in_specs=[
            pl.BlockSpec((BR, C), lambda i: (i, 0)),
            pl.BlockSpec((C, C), lambda i: (0, 0)),
            pl.BlockSpec((1, C), lambda i: (0, 0)),
            pl.BlockSpec((1, C), lambda i: (0, 0)),
        ],
        out_specs=_TRANSFORM_OUT_SPECS,
        out_shape=_TRANSFORM_OUT_SHAPE,
    )(x, w, asrc, adst)


def _tc_combine_body(plo_ref, phi_ref, d_ref, b_ref,
                     w_ref, asrc_ref, adst_ref,
                     hlo_ref, hhi_ref, as_ref, ad_ref):
    den = d_ref[...] + 1e-16
    o = jnp.concatenate([plo_ref[...], phi_ref[...]], axis=1) / den + b_ref[...]
    o = jnp.where(o > 0.0, o, jnp.exp(jnp.minimum(o, 0.0)) - 1.0)
    h = jnp.dot(o, w_ref[...], preferred_element_type=jnp.float32)
    _split_out(h, hlo_ref, hhi_ref, asrc_ref[...], adst_ref[...],
               as_ref, ad_ref)


def _tc_combine(plo, phi, d, b, w, asrc, adst):
    return pl.pallas_call(
        _tc_combine_body,
        grid=(GRID,),
        in_specs=[
            pl.BlockSpec((BR, H), lambda i: (i, 0)),
            pl.BlockSpec((BR, H), lambda i: (i, 0)),
            pl.BlockSpec((BR, 1), lambda i: (i, 0)),
            pl.BlockSpec((1, C), lambda i: (0, 0)),
            pl.BlockSpec((C, C), lambda i: (0, 0)),
            pl.BlockSpec((1, C), lambda i: (0, 0)),
            pl.BlockSpec((1, C), lambda i: (0, 0)),
        ],
        out_specs=_TRANSFORM_OUT_SPECS,
        out_shape=_TRANSFORM_OUT_SHAPE,
    )(plo, phi, d, b, w, asrc, adst)


def _tc_final_body(plo_ref, phi_ref, d_ref, b_ref, o_ref):
    den = d_ref[...] + 1e-16
    o_ref[...] = (jnp.concatenate([plo_ref[...], phi_ref[...]], axis=1) / den
                  + b_ref[...])


def _tc_final(plo, phi, d, b):
    return pl.pallas_call(
        _tc_final_body,
        grid=(GRID,),
        in_specs=[
            pl.BlockSpec((BR, H), lambda i: (i, 0)),
            pl.BlockSpec((BR, H), lambda i: (i, 0)),
            pl.BlockSpec((BR, 1), lambda i: (i, 0)),
            pl.BlockSpec((1, C), lambda i: (0, 0)),
        ],
        out_specs=pl.BlockSpec((BR, C), lambda i: (i, 0)),
        out_shape=jax.ShapeDtypeStruct((N, C), jnp.float32),
    )(plo, phi, d, b)


def kernel(x, edge_index, W1, att_src1, att_dst1, b1,
           W2, att_src2, att_dst2, b2):
    ei = edge_index.astype(jnp.int32)
    src_r = ei[0].reshape(NS, NBLK, G)
    src = jnp.concatenate([src_r, src_r + N], axis=0)
    dst = ei[1].reshape(NS, NBLK, G)

    hlo1, hhi1, as1, ad1 = _tc_transform(
        x, W1, att_src1.reshape(1, C), att_dst1.reshape(1, C))
    olo, ohi, d1 = _sc_edge(src, dst, as1.reshape(-1), ad1.reshape(-1),
                            jnp.concatenate([hlo1, hhi1], axis=0))

    hlo2, hhi2, as2, ad2 = _tc_combine(
        olo, ohi, d1[:N].reshape(N, 1), b1.reshape(1, C),
        W2, att_src2.reshape(1, C), att_dst2.reshape(1, C))
    qlo, qhi, d2 = _sc_edge(src, dst, as2.reshape(-1), ad2.reshape(-1),
                            jnp.concatenate([hlo2, hhi2], axis=0))

    return _tc_final(qlo, qhi, d2[:N].reshape(N, 1), b2.reshape(1, C))
